# baseline (speedup 1.0000x reference)
import functools

import jax
import jax.numpy as jnp
from jax import lax
from jax.experimental import pallas as pl
from jax.experimental.pallas import tpu as pltpu
from jax.experimental.pallas import tpu_sc as plsc

_H = 224
_W = 224
_HW = _H * _W
_N = 2
_C = 384
_NC = 2
_NS = 16
_NW = _NC * _NS
_CPW = (_N * _C) // _NW
_S = 3584
_NCHUNK = _HW // _S


def _prep_body(gx_ref, gy_ref, idx_ref, w_ref):
    gx = gx_ref[...]
    gy = gy_ref[...]
    ix = ((gx + 1.0) * _W - 1.0) * 0.5
    iy = ((gy + 1.0) * _H - 1.0) * 0.5
    x0f = jnp.clip(jnp.floor(ix), -2.0, float(_W))
    y0f = jnp.clip(jnp.floor(iy), -2.0, float(_H))
    wx1 = ix - x0f
    wx0 = 1.0 - wx1
    wy1 = iy - y0f
    wy0 = 1.0 - wy1
    x0 = x0f.astype(jnp.int32)
    y0 = y0f.astype(jnp.int32)
    inx = ((x0 >= 0) & (x0 <= _W - 2)).astype(jnp.float32)
    iny = ((y0 >= 0) & (y0 <= _H - 2)).astype(jnp.float32)
    wl = wx0 * inx + wx1 * (x0 == -1)
    wr = wx1 * inx + wx0 * (x0 == _W - 1)
    wt = wy0 * iny + wy1 * (y0 == -1)
    wb = wy1 * iny + wy0 * (y0 == _H - 1)
    xb = jnp.clip(x0, 0, _W - 2)
    yb = jnp.clip(y0, 0, _H - 2)
    idx_ref[...] = yb * _W + xb
    w_ref[...] = jnp.stack([wt * wl, wt * wr, wb * wl, wb * wr], axis=1)


def _prep(gx, gy):
    return pl.pallas_call(
        _prep_body,
        out_shape=(
            jax.ShapeDtypeStruct((_N, _HW), jnp.int32),
            jax.ShapeDtypeStruct((_N, 4, _HW), jnp.float32),
        ),
    )(gx, gy)


_mesh = plsc.VectorSubcoreMesh(core_axis_name="c", subcore_axis_name="s")


@functools.partial(
    pl.kernel,
    out_type=jax.ShapeDtypeStruct((_N * _C, _HW), jnp.float32),
    mesh=_mesh,
    compiler_params=pltpu.CompilerParams(needs_layout_passes=False),
    scratch_types=[
        pltpu.VMEM((_HW,), jnp.float32),
        pltpu.VMEM((_HW,), jnp.float32),
        pltpu.VMEM((_S,), jnp.int32),
        pltpu.VMEM((_S,), jnp.float32),
        pltpu.VMEM((_S,), jnp.float32),
        pltpu.VMEM((_S,), jnp.float32),
        pltpu.VMEM((_S,), jnp.float32),
        pltpu.VMEM((_S,), jnp.float32),
        pltpu.VMEM((_S,), jnp.float32),
        pltpu.SemaphoreType.DMA,
    ],
)
def _sc_sample(
    x_hbm, idx_hbm, w_hbm, out_hbm,
    plane0_v, plane1_v, idxc_v, w00c_v, w01c_v, w10c_v, w11c_v,
    out0c_v, out1c_v, sem,
):
    wid = lax.axis_index("s") * _NC + lax.axis_index("c")
    base_plane = wid * _CPW
    b = base_plane // _C

    def pair_body(pp, _):
        p0 = base_plane + 2 * pp
        cp0 = pltpu.async_copy(x_hbm.at[p0], plane0_v, sem)
        cp1 = pltpu.async_copy(x_hbm.at[p0 + 1], plane1_v, sem)
        cp0.wait()
        cp1.wait()

        def chunk_body(c, _):
            off = c * _S
            cps = [
                pltpu.async_copy(idx_hbm.at[b, pl.ds(off, _S)], idxc_v, sem),
                pltpu.async_copy(w_hbm.at[b, 0, pl.ds(off, _S)], w00c_v, sem),
                pltpu.async_copy(w_hbm.at[b, 1, pl.ds(off, _S)], w01c_v, sem),
                pltpu.async_copy(w_hbm.at[b, 2, pl.ds(off, _S)], w10c_v, sem),
                pltpu.async_copy(w_hbm.at[b, 3, pl.ds(off, _S)], w11c_v, sem),
            ]
            for cp in cps:
                cp.wait()

            def vec_body(i, _):
                o = i * 16
                iv = idxc_v[pl.ds(o, 16)]
                w00 = w00c_v[pl.ds(o, 16)]
                w01 = w01c_v[pl.ds(o, 16)]
                w10 = w10c_v[pl.ds(o, 16)]
                w11 = w11c_v[pl.ds(o, 16)]
                iv1 = iv + 1
                iv2 = iv + _W
                iv3 = iv + (_W + 1)
                for pk, ok in ((plane0_v, out0c_v), (plane1_v, out1c_v)):
                    v00 = plsc.load_gather(pk, [iv])
                    v01 = plsc.load_gather(pk, [iv1])
                    v10 = plsc.load_gather(pk, [iv2])
                    v11 = plsc.load_gather(pk, [iv3])
                    ok[pl.ds(o, 16)] = (
                        v00 * w00 + v01 * w01 + v10 * w10 + v11 * w11
                    )
                return 0

            lax.fori_loop(0, _S // 16, vec_body, 0)
            co0 = pltpu.async_copy(out0c_v, out_hbm.at[p0, pl.ds(off, _S)], sem)
            co1 = pltpu.async_copy(out1c_v, out_hbm.at[p0 + 1, pl.ds(off, _S)], sem)
            co0.wait()
            co1.wait()
            return 0

        lax.fori_loop(0, _NCHUNK, chunk_body, 0)
        return 0

    lax.fori_loop(0, _CPW // 2, pair_body, 0)


def kernel(x, grid):
    gx = grid[..., 0].reshape(_N, _HW)
    gy = grid[..., 1].reshape(_N, _HW)
    idx, w4 = _prep(gx, gy)
    out_flat = _sc_sample(x.reshape(_N * _C, _HW), idx, w4)
    return out_flat.reshape(_N, _C, _H, _W)


# speedup vs baseline: 1.0001x; 1.0001x over previous
import functools

import jax
import jax.numpy as jnp
from jax import lax
from jax.experimental import pallas as pl
from jax.experimental.pallas import tpu as pltpu
from jax.experimental.pallas import tpu_sc as plsc

_H = 224
_W = 224
_HW = _H * _W
_N = 2
_C = 384
_NC = 2
_NS = 16
_NW = _NC * _NS
_CPW = (_N * _C) // _NW
_S = 1792
_NCHUNK = _HW // _S


def _prep_body(gx_ref, gy_ref, iw_ref):
    gx = gx_ref[...]
    gy = gy_ref[...]
    ix = ((gx + 1.0) * _W - 1.0) * 0.5
    iy = ((gy + 1.0) * _H - 1.0) * 0.5
    x0f = jnp.clip(jnp.floor(ix), -2.0, float(_W))
    y0f = jnp.clip(jnp.floor(iy), -2.0, float(_H))
    wx1 = ix - x0f
    wx0 = 1.0 - wx1
    wy1 = iy - y0f
    wy0 = 1.0 - wy1
    x0 = x0f.astype(jnp.int32)
    y0 = y0f.astype(jnp.int32)
    inx = ((x0 >= 0) & (x0 <= _W - 2)).astype(jnp.float32)
    iny = ((y0 >= 0) & (y0 <= _H - 2)).astype(jnp.float32)
    wl = wx0 * inx + wx1 * (x0 == -1)
    wr = wx1 * inx + wx0 * (x0 == _W - 1)
    wt = wy0 * iny + wy1 * (y0 == -1)
    wb = wy1 * iny + wy0 * (y0 == _H - 1)
    xb = jnp.clip(x0, 0, _W - 2)
    yb = jnp.clip(y0, 0, _H - 2)
    idx_f = lax.bitcast_convert_type(yb * _W + xb, jnp.float32)
    rows = [idx_f, wt * wl, wt * wr, wb * wl, wb * wr]
    iw_ref[...] = jnp.concatenate(
        [r.reshape(_N, _NCHUNK, _S) for r in rows], axis=-1
    )


def _prep(gx, gy):
    return pl.pallas_call(
        _prep_body,
        out_shape=jax.ShapeDtypeStruct((_N, _NCHUNK, 5 * _S), jnp.float32),
    )(gx, gy)


_mesh = plsc.VectorSubcoreMesh(core_axis_name="c", subcore_axis_name="s")


@functools.partial(
    pl.kernel,
    out_type=jax.ShapeDtypeStruct((_N * _C, _HW), jnp.float32),
    mesh=_mesh,
    compiler_params=pltpu.CompilerParams(needs_layout_passes=False),
    scratch_types=[
        pltpu.VMEM((_HW,), jnp.float32),
        pltpu.VMEM((_HW,), jnp.float32),
        (pltpu.VMEM((5 * _S,), jnp.float32),
         pltpu.VMEM((5 * _S,), jnp.float32)),
        pltpu.VMEM((2, 2, _S), jnp.float32),
        pltpu.SemaphoreType.DMA,
        (pltpu.SemaphoreType.DMA, pltpu.SemaphoreType.DMA),
        (pltpu.SemaphoreType.DMA, pltpu.SemaphoreType.DMA),
    ],
)
def _sc_sample(
    x_hbm, iw_hbm, out_hbm,
    plane0_v, plane1_v, iw_v, out_v, sem_pl, sem_iw, sem_out,
):
    wid = lax.axis_index("s") * _NC + lax.axis_index("c")
    base_plane = wid * _CPW
    b = base_plane // _C

    def iw_copy(c, buf):
        return pltpu.make_async_copy(iw_hbm.at[b, c], iw_v[buf], sem_iw[buf])

    def pair_body(pp, _):
        p0 = base_plane + 2 * pp
        cp0 = pltpu.async_copy(x_hbm.at[p0], plane0_v, sem_pl)
        cp1 = pltpu.async_copy(x_hbm.at[p0 + 1], plane1_v, sem_pl)
        iw_copy(0, 0).start()
        iw_copy(1, 1).start()
        cp0.wait()
        cp1.wait()

        def chunk2_body(cc, _):
            for buf in (0, 1):
                c = cc * 2 + buf
                off = c * _S
                iw_copy(c, buf).wait()

                @pl.when(c >= 2)
                def _():
                    pltpu.make_async_copy(
                        out_v.at[buf, 0], out_hbm.at[p0, pl.ds(off, _S)],
                        sem_out[buf],
                    ).wait()
                    pltpu.make_async_copy(
                        out_v.at[buf, 1], out_hbm.at[p0, pl.ds(off, _S)],
                        sem_out[buf],
                    ).wait()

                iwb = iw_v[buf]

                def vec_body(i, _):
                    o = i * 16
                    iv = plsc.bitcast(iwb[pl.ds(o, 16)], jnp.int32)
                    w00 = iwb[pl.ds(_S + o, 16)]
                    w01 = iwb[pl.ds(2 * _S + o, 16)]
                    w10 = iwb[pl.ds(3 * _S + o, 16)]
                    w11 = iwb[pl.ds(4 * _S + o, 16)]
                    iv1 = iv + 1
                    iv2 = iv + _W
                    iv3 = iv + (_W + 1)
                    for k, pk in ((0, plane0_v), (1, plane1_v)):
                        v00 = plsc.load_gather(pk, [iv])
                        v01 = plsc.load_gather(pk, [iv1])
                        v10 = plsc.load_gather(pk, [iv2])
                        v11 = plsc.load_gather(pk, [iv3])
                        out_v[buf, k, pl.ds(o, 16)] = (
                            v00 * w00 + v01 * w01 + v10 * w10 + v11 * w11
                        )
                    return 0

                lax.fori_loop(0, _S // 16, vec_body, 0)

                @pl.when(c + 2 < _NCHUNK)
                def _():
                    iw_copy(c + 2, buf).start()

                pltpu.async_copy(
                    out_v.at[buf, 0], out_hbm.at[p0, pl.ds(off, _S)], sem_out[buf]
                )
                pltpu.async_copy(
                    out_v.at[buf, 1], out_hbm.at[p0 + 1, pl.ds(off, _S)],
                    sem_out[buf],
                )
            return 0

        lax.fori_loop(0, _NCHUNK // 2, chunk2_body, 0)
        for buf in (0, 1):
            pltpu.make_async_copy(
                out_v.at[buf, 0], out_hbm.at[p0, pl.ds(0, _S)], sem_out[buf]
            ).wait()
            pltpu.make_async_copy(
                out_v.at[buf, 1], out_hbm.at[p0, pl.ds(0, _S)], sem_out[buf]
            ).wait()
        return 0

    lax.fori_loop(0, _CPW // 2, pair_body, 0)


def kernel(x, grid):
    gx = grid[..., 0].reshape(_N, _HW)
    gy = grid[..., 1].reshape(_N, _HW)
    iw = _prep(gx, gy)
    out_flat = _sc_sample(x.reshape(_N * _C, _HW), iw)
    return out_flat.reshape(_N, _C, _H, _W)


# speedup vs baseline: 1.4953x; 1.4952x over previous
import functools

import jax
import jax.numpy as jnp
from jax import lax
from jax.experimental import pallas as pl
from jax.experimental.pallas import tpu as pltpu
from jax.experimental.pallas import tpu_sc as plsc

_H = 224
_W = 224
_HW = _H * _W
_N = 2
_C = 384
_NC = 2
_NS = 16
_NW = _NC * _NS
_CPW = (_N * _C) // _NW
_S = 1792
_NCHUNK = _HW // _S


def _prep_body(gx_ref, gy_ref, iw_ref):
    gx = gx_ref[...]
    gy = gy_ref[...]
    ix = ((gx + 1.0) * _W - 1.0) * 0.5
    iy = ((gy + 1.0) * _H - 1.0) * 0.5
    x0f = jnp.clip(jnp.floor(ix), -2.0, float(_W))
    y0f = jnp.clip(jnp.floor(iy), -2.0, float(_H))
    wx1 = ix - x0f
    wx0 = 1.0 - wx1
    wy1 = iy - y0f
    wy0 = 1.0 - wy1
    x0 = x0f.astype(jnp.int32)
    y0 = y0f.astype(jnp.int32)
    inx = ((x0 >= 0) & (x0 <= _W - 2)).astype(jnp.float32)
    iny = ((y0 >= 0) & (y0 <= _H - 2)).astype(jnp.float32)
    wl = wx0 * inx + wx1 * (x0 == -1)
    wr = wx1 * inx + wx0 * (x0 == _W - 1)
    wt = wy0 * iny + wy1 * (y0 == -1)
    wb = wy1 * iny + wy0 * (y0 == _H - 1)
    xb = jnp.clip(x0, 0, _W - 2)
    yb = jnp.clip(y0, 0, _H - 2)
    idx_f = lax.bitcast_convert_type(yb * _W + xb, jnp.float32)
    rows = [idx_f, wt * wl, wt * wr, wb * wl, wb * wr]
    iw_ref[...] = jnp.concatenate(
        [r.reshape(_N, _NCHUNK, _S) for r in rows], axis=-1
    )


def _prep(gx, gy):
    return pl.pallas_call(
        _prep_body,
        out_shape=jax.ShapeDtypeStruct((_N, _NCHUNK, 5 * _S), jnp.float32),
    )(gx, gy)


_mesh = plsc.VectorSubcoreMesh(core_axis_name="c", subcore_axis_name="s")


@functools.partial(
    pl.kernel,
    out_type=jax.ShapeDtypeStruct((_N * _C, _HW), jnp.float32),
    mesh=_mesh,
    compiler_params=pltpu.CompilerParams(needs_layout_passes=False),
    scratch_types=[
        pltpu.VMEM((_HW,), jnp.float32),
        pltpu.VMEM((_HW,), jnp.float32),
        (pltpu.VMEM((5 * _S,), jnp.float32),
         pltpu.VMEM((5 * _S,), jnp.float32)),
        pltpu.VMEM((2, 2, _S), jnp.float32),
        pltpu.SemaphoreType.DMA,
        (pltpu.SemaphoreType.DMA, pltpu.SemaphoreType.DMA),
        (pltpu.SemaphoreType.DMA, pltpu.SemaphoreType.DMA),
    ],
)
def _sc_sample(
    x_hbm, iw_hbm, out_hbm,
    plane0_v, plane1_v, iw_v, out_v, sem_pl, sem_iw, sem_out,
):
    wid = lax.axis_index("s") * _NC + lax.axis_index("c")
    base_plane = wid * _CPW
    b = base_plane // _C

    def iw_copy(c, buf):
        return pltpu.make_async_copy(iw_hbm.at[b, c], iw_v[buf], sem_iw[buf])

    def pair_body(pp, _):
        p0 = base_plane + 2 * pp
        cp0 = pltpu.async_copy(x_hbm.at[p0], plane0_v, sem_pl)
        cp1 = pltpu.async_copy(x_hbm.at[p0 + 1], plane1_v, sem_pl)
        iw_copy(0, 0).start()
        iw_copy(1, 1).start()
        cp0.wait()
        cp1.wait()

        def chunk2_body(cc, _):
            for buf in (0, 1):
                c = cc * 2 + buf
                off = c * _S
                iw_copy(c, buf).wait()

                @pl.when(c >= 2)
                def _():
                    pltpu.make_async_copy(
                        out_v.at[buf, 0], out_hbm.at[p0, pl.ds(off, _S)],
                        sem_out[buf],
                    ).wait()
                    pltpu.make_async_copy(
                        out_v.at[buf, 1], out_hbm.at[p0, pl.ds(off, _S)],
                        sem_out[buf],
                    ).wait()

                iwb = iw_v[buf]

                @plsc.parallel_loop(0, _S, step=16, unroll=4)
                def vec_body(o):
                    iv = plsc.bitcast(iwb[pl.ds(o, 16)], jnp.int32)
                    w00 = iwb[pl.ds(_S + o, 16)]
                    w01 = iwb[pl.ds(2 * _S + o, 16)]
                    w10 = iwb[pl.ds(3 * _S + o, 16)]
                    w11 = iwb[pl.ds(4 * _S + o, 16)]
                    iv1 = iv + 1
                    iv2 = iv + _W
                    iv3 = iv + (_W + 1)
                    for k, pk in ((0, plane0_v), (1, plane1_v)):
                        v00 = plsc.load_gather(pk, [iv])
                        v01 = plsc.load_gather(pk, [iv1])
                        v10 = plsc.load_gather(pk, [iv2])
                        v11 = plsc.load_gather(pk, [iv3])
                        out_v[buf, k, pl.ds(o, 16)] = (
                            v00 * w00 + v01 * w01 + v10 * w10 + v11 * w11
                        )

                @pl.when(c + 2 < _NCHUNK)
                def _():
                    iw_copy(c + 2, buf).start()

                pltpu.async_copy(
                    out_v.at[buf, 0], out_hbm.at[p0, pl.ds(off, _S)], sem_out[buf]
                )
                pltpu.async_copy(
                    out_v.at[buf, 1], out_hbm.at[p0 + 1, pl.ds(off, _S)],
                    sem_out[buf],
                )
            return 0

        lax.fori_loop(0, _NCHUNK // 2, chunk2_body, 0)
        for buf in (0, 1):
            pltpu.make_async_copy(
                out_v.at[buf, 0], out_hbm.at[p0, pl.ds(0, _S)], sem_out[buf]
            ).wait()
            pltpu.make_async_copy(
                out_v.at[buf, 1], out_hbm.at[p0, pl.ds(0, _S)], sem_out[buf]
            ).wait()
        return 0

    lax.fori_loop(0, _CPW // 2, pair_body, 0)


def kernel(x, grid):
    gx = grid[..., 0].reshape(_N, _HW)
    gy = grid[..., 1].reshape(_N, _HW)
    iw = _prep(gx, gy)
    out_flat = _sc_sample(x.reshape(_N * _C, _HW), iw)
    return out_flat.reshape(_N, _C, _H, _W)
